# Initial kernel scaffold; baseline (speedup 1.0000x reference)
import functools

import jax
import jax.numpy as jnp
from jax import lax
from jax.experimental import pallas as pl
from jax.experimental.pallas import tpu as pltpu
from jax.experimental.pallas import tpu_sc as plsc

NC = 2
NS = 16
NW = NC * NS
LANE = 16


def _mesh():
    return plsc.VectorSubcoreMesh(
        core_axis_name="c", subcore_axis_name="s", num_cores=NC, num_subcores=NS
    )


def _make_hist(ROWS, RPT, TAIL, G, NCHUNK, NPAD):
    def body(row2d, col2d, out_parts, idx_v, hist_v):
        cid = lax.axis_index("c")
        sid = lax.axis_index("s")
        w = sid * NC + cid
        ones = jnp.full((LANE,), 1.0, jnp.float32)
        zeros = jnp.zeros((LANE,), jnp.float32)

        for ph, src in enumerate((row2d, col2d)):
            def zero_body(i, _):
                hist_v[pl.ds(i * LANE, LANE)] = zeros
                return 0

            lax.fori_loop(0, NPAD // LANE, zero_body, 0)

            def chunk_body(c, _):
                start = w * RPT + c * G
                pltpu.sync_copy(src.at[pl.ds(start, G)], idx_v)

                def row_body(r, _):
                    for k in range(128 // LANE):
                        idx16 = idx_v[r, pl.ds(k * LANE, LANE)]
                        plsc.addupdate_scatter(hist_v, [idx16], ones)
                    return 0

                lax.fori_loop(0, G, row_body, 0)
                return 0

            lax.fori_loop(0, NCHUNK, chunk_body, 0)

            if TAIL:
                @pl.when(w < TAIL)
                def _():
                    pltpu.sync_copy(
                        src.at[pl.ds(RPT * NW + w, 1)], idx_v.at[pl.ds(0, 1)]
                    )
                    for k in range(128 // LANE):
                        idx16 = idx_v[0, pl.ds(k * LANE, LANE)]
                        plsc.addupdate_scatter(hist_v, [idx16], ones)

            pltpu.sync_copy(hist_v, out_parts.at[ph, w])

    kern = functools.partial(
        pl.kernel,
        out_type=jax.ShapeDtypeStruct((2, NW, NPAD), jnp.float32),
        mesh=_mesh(),
        scratch_types=[
            pltpu.VMEM((G, 128), jnp.int32),
            pltpu.VMEM((NPAD,), jnp.float32),
        ],
        name="sc_degree_hist",
    )
    return kern(body)


def _make_gs(F, ROWS, RPT, TAIL, G, NCHUNK, NPAD):
    NPT = NPAD // NS
    ZR = NPT // 4

    def body(row2d, col2d, table, out, rowb, colb, rows_v, zbuf, acc_sh,
             gsem, ssem):
        cid = lax.axis_index("c")
        sid = lax.axis_index("s")
        w = sid * NC + cid
        zeros = jnp.zeros((LANE,), jnp.float32)

        def zb_body(r, _):
            zbuf[r, :] = zeros
            return 0

        lax.fori_loop(0, ZR, zb_body, 0)

        def do_row(j):
            pltpu.async_copy(
                table.at[rowb.at[j]], rows_v.at[pl.ds(j * 128, 128)], gsem
            ).wait()
            pltpu.async_copy(
                rows_v.at[pl.ds(j * 128, 128)], acc_sh.at[colb.at[j]], ssem,
                add=True,
            ).wait()

        def f_body(f, _):
            for t in range(4):
                pltpu.sync_copy(
                    zbuf, acc_sh.at[pl.ds(sid * NPT + t * ZR, ZR)]
                )
            plsc.subcore_barrier()

            def chunk_body(c, _):
                start = w * RPT + c * G
                pltpu.sync_copy(row2d.at[pl.ds(start, G)], rowb)
                pltpu.sync_copy(col2d.at[pl.ds(start, G)], colb)
                if F > 1:
                    off = f * NPAD

                    def add_body(r, _):
                        for k in range(128 // LANE):
                            sl = pl.ds(k * LANE, LANE)
                            rowb[r, sl] = rowb[r, sl] + off
                        return 0

                    lax.fori_loop(0, G, add_body, 0)
                for j in range(G):
                    do_row(j)
                return 0

            lax.fori_loop(0, NCHUNK, chunk_body, 0)

            if TAIL:
                @pl.when(w < TAIL)
                def _():
                    pltpu.sync_copy(
                        row2d.at[pl.ds(RPT * NW + w, 1)], rowb.at[pl.ds(0, 1)]
                    )
                    pltpu.sync_copy(
                        col2d.at[pl.ds(RPT * NW + w, 1)], colb.at[pl.ds(0, 1)]
                    )
                    if F > 1:
                        off = f * NPAD
                        for k in range(128 // LANE):
                            sl = pl.ds(k * LANE, LANE)
                            rowb[0, sl] = rowb[0, sl] + off
                    do_row(0)

            plsc.subcore_barrier()
            pltpu.sync_copy(
                acc_sh.at[pl.ds(sid * NPT, NPT)],
                out.at[pl.ds(f * (NC * NPAD) + cid * NPAD + sid * NPT, NPT)],
            )
            plsc.subcore_barrier()
            return 0

        lax.fori_loop(0, F, f_body, 0)

    kern = functools.partial(
        pl.kernel,
        out_type=jax.ShapeDtypeStruct((F * NC * NPAD, 16), jnp.float32),
        mesh=_mesh(),
        scratch_types=[
            pltpu.VMEM((G, 128), jnp.int32),
            pltpu.VMEM((G, 128), jnp.int32),
            pltpu.VMEM((G * 128, 16), jnp.float32),
            pltpu.VMEM((NPAD // NS // 4, 16), jnp.float32),
            pltpu.VMEM_SHARED((NPAD, 16), jnp.float32),
            pltpu.SemaphoreType.DMA,
            pltpu.SemaphoreType.DMA,
        ],
        name=f"sc_gather_scatter_f{F}",
    )
    return kern(body)


def _dense_a(parts, xr, NPAD, BN):
    grid = (NPAD // BN,)

    def body(parts_ref, x_ref, dinv_ref, indeg_ref, ds_ref, y16t_ref):
        p = parts_ref[...]
        ds = jnp.sum(p[0], axis=0, keepdims=True)
        indeg = jnp.sum(p[1], axis=0, keepdims=True)
        dinv = lax.rsqrt(indeg + 1.0)
        xv = x_ref[...]
        y1 = dinv * xv
        y2 = dinv * ds
        dinv_ref[...] = dinv
        indeg_ref[...] = indeg
        ds_ref[...] = ds
        y16t_ref[...] = jnp.concatenate(
            [y1, y2, jnp.zeros((14, BN), jnp.float32)], axis=0
        )

    row = jax.ShapeDtypeStruct((1, NPAD), jnp.float32)
    return pl.pallas_call(
        body,
        grid=grid,
        in_specs=[
            pl.BlockSpec((2, NW, BN), lambda i: (0, 0, i)),
            pl.BlockSpec((1, BN), lambda i: (0, i)),
        ],
        out_specs=[
            pl.BlockSpec((1, BN), lambda i: (0, i)),
            pl.BlockSpec((1, BN), lambda i: (0, i)),
            pl.BlockSpec((1, BN), lambda i: (0, i)),
            pl.BlockSpec((16, BN), lambda i: (0, i)),
        ],
        out_shape=[row, row, row, jax.ShapeDtypeStruct((16, NPAD), jnp.float32)],
    )(parts, xr)


def _dense_b(uvt, dinv, xr, dsr, W1T, b1c, NPAD, BN):
    grid = (NPAD // BN,)

    def body(uvt_ref, dinv_ref, x_ref, ds_ref, w1t_ref, b1_ref, a1t_ref):
        uvt_v = uvt_ref[...]
        U = uvt_v[0, 0:1] + uvt_v[1, 0:1]
        V = uvt_v[0, 1:2] + uvt_v[1, 1:2]
        dinv = dinv_ref[...]
        P = dinv * (U + dinv * x_ref[...])
        Q = dinv * (V + dinv * ds_ref[...])
        w = w1t_ref[...]
        pre = w[:, 0:1] * P + w[:, 1:2] * Q + b1_ref[...]
        a1t_ref[...] = jnp.maximum(pre, 0.0)

    return pl.pallas_call(
        body,
        grid=grid,
        in_specs=[
            pl.BlockSpec((NC, 16, BN), lambda i: (0, 0, i)),
            pl.BlockSpec((1, BN), lambda i: (0, i)),
            pl.BlockSpec((1, BN), lambda i: (0, i)),
            pl.BlockSpec((1, BN), lambda i: (0, i)),
            pl.BlockSpec((128, 2), lambda i: (0, 0)),
            pl.BlockSpec((128, 1), lambda i: (0, 0)),
        ],
        out_specs=pl.BlockSpec((128, BN), lambda i: (0, i)),
        out_shape=jax.ShapeDtypeStruct((128, NPAD), jnp.float32),
    )(uvt, dinv, xr, dsr, W1T, b1c)


def _dense_c(nb0t, nb1t, a1t, indeg, W2lT, W2rT, Wv1T, Wv2T, b2lc, bv1c,
             bv2c, N, NPAD, BN):
    grid = (NPAD // BN,)

    def body(nb0_ref, nb1_ref, a1t_ref, indeg_ref, w2l_ref, w2r_ref,
             wv1_ref, wv2_ref, b2l_ref, bv1_ref, bv2_ref, out_ref):
        i = pl.program_id(0)
        cnt = indeg_ref[...]
        recip = 1.0 / jnp.maximum(cnt, 1.0)
        mean_t = (nb0_ref[...] + nb1_ref[...]) * recip
        a1t_v = a1t_ref[...]
        a2t = (
            jnp.dot(w2l_ref[...], mean_t, preferred_element_type=jnp.float32)
            + b2l_ref[...]
            + jnp.dot(w2r_ref[...], a1t_v, preferred_element_type=jnp.float32)
        )
        h = jnp.maximum(
            jnp.dot(wv1_ref[...], a2t, preferred_element_type=jnp.float32)
            + bv1_ref[...],
            0.0,
        )
        v = jnp.dot(wv2_ref[...], h, preferred_element_type=jnp.float32) \
            + bv2_ref[...]
        colid = i * BN + lax.broadcasted_iota(jnp.int32, (1, BN), 1)
        v = jnp.where(colid < N, v, 0.0)
        s = jnp.sum(v)
        prev = jnp.where(i == 0, 0.0, out_ref[0, 0])
        out_ref[0, 0] = prev + s

    return pl.pallas_call(
        body,
        grid=grid,
        in_specs=[
            pl.BlockSpec((128, BN), lambda i: (0, i)),
            pl.BlockSpec((128, BN), lambda i: (0, i)),
            pl.BlockSpec((128, BN), lambda i: (0, i)),
            pl.BlockSpec((1, BN), lambda i: (0, i)),
            pl.BlockSpec((128, 128), lambda i: (0, 0)),
            pl.BlockSpec((128, 128), lambda i: (0, 0)),
            pl.BlockSpec((64, 128), lambda i: (0, 0)),
            pl.BlockSpec((1, 64), lambda i: (0, 0)),
            pl.BlockSpec((128, 1), lambda i: (0, 0)),
            pl.BlockSpec((64, 1), lambda i: (0, 0)),
            pl.BlockSpec((1, 1), lambda i: (0, 0)),
        ],
        out_specs=pl.BlockSpec((1, 1), lambda i: (0, 0)),
        out_shape=jax.ShapeDtypeStruct((1, 1), jnp.float32),
    )(nb0t, nb1t, a1t, indeg, W2lT, W2rT, Wv1T, Wv2T, b2lc, bv1c, bv2c)


def kernel(x, edge_index, W1, b1, W2l, b2l, W2r, Wv1, bv1, Wv2, bv2):
    N = x.shape[0]
    E = edge_index.shape[1]
    assert E % 128 == 0
    NPAD = ((N + 2047) // 2048) * 2048
    BN = 2048
    ROWS = E // 128
    RPT = ROWS // NW
    TAIL = ROWS % NW
    G = max(g for g in range(1, 25) if RPT % g == 0)
    NCHUNK = RPT // G
    F = 8

    row2d = edge_index[0].reshape(ROWS, 128)
    col2d = edge_index[1].reshape(ROWS, 128)
    xr = jnp.pad(x[:, 0], (0, NPAD - N)).reshape(1, NPAD)

    parts = _make_hist(ROWS, RPT, TAIL, G, NCHUNK, NPAD)(row2d, col2d)
    dinv, indeg, dsr, y16t = _dense_a(parts, xr, NPAD, BN)

    y16 = y16t.T
    uv = _make_gs(1, ROWS, RPT, TAIL, G, NCHUNK, NPAD)(row2d, col2d, y16)
    uvt = uv.reshape(NC, NPAD, 16).transpose(0, 2, 1)

    a1t = _dense_b(uvt, dinv, xr, dsr, W1.T, b1.reshape(128, 1), NPAD, BN)
    a1s = a1t.reshape(F, 16, NPAD).transpose(0, 2, 1).reshape(F * NPAD, 16)

    nbf = _make_gs(F, ROWS, RPT, TAIL, G, NCHUNK, NPAD)(row2d, col2d, a1s)
    nb4 = nbf.reshape(F, NC, NPAD, 16)
    nb0t = nb4[:, 0].transpose(0, 2, 1).reshape(128, NPAD)
    nb1t = nb4[:, 1].transpose(0, 2, 1).reshape(128, NPAD)

    out = _dense_c(
        nb0t, nb1t, a1t, indeg, W2l.T, W2r.T, Wv1.T, Wv2.T,
        b2l.reshape(128, 1), bv1.reshape(64, 1), bv2.reshape(1, 1),
        N, NPAD, BN,
    )
    return out[0, 0]


# speedup vs baseline: 10.6309x; 10.6309x over previous
import functools

import jax
import jax.numpy as jnp
from jax import lax
from jax.experimental import pallas as pl
from jax.experimental.pallas import tpu as pltpu
from jax.experimental.pallas import tpu_sc as plsc

NC = 2
NS = 16
NW = NC * NS
LANE = 16


def _mesh():
    return plsc.VectorSubcoreMesh(
        core_axis_name="c", subcore_axis_name="s", num_cores=NC, num_subcores=NS
    )


def _make_hist(GPT, GTAIL, GG, NCHUNK, NPAD):
    G = GG * 8

    def body(row2d, col2d, out_src, out_dst, idx_v, hist_v):
        cid = lax.axis_index("c")
        sid = lax.axis_index("s")
        w = sid * NC + cid
        ones = jnp.full((LANE,), 1.0, jnp.float32)
        zeros = jnp.zeros((LANE,), jnp.float32)

        def scatter_row(r):
            for k in range(128 // LANE):
                idx16 = idx_v[r, pl.ds(k * LANE, LANE)]
                plsc.addupdate_scatter(hist_v, [idx16], ones)

        for ph, (src, dst) in enumerate(((row2d, out_src), (col2d, out_dst))):
            def zero_body(i, _):
                hist_v[pl.ds(i * LANE, LANE)] = zeros
                return 0

            lax.fori_loop(0, NPAD // LANE, zero_body, 0)

            def chunk_body(c, _):
                start = (w * GPT + c * GG) * 8
                pltpu.sync_copy(src.at[pl.ds(start, G)], idx_v)

                def row_body(r, _):
                    scatter_row(r)
                    return 0

                lax.fori_loop(0, G, row_body, 0)
                return 0

            lax.fori_loop(0, NCHUNK, chunk_body, 0)

            if GTAIL:
                @pl.when(w < GTAIL)
                def _():
                    start = (GPT * NW + w) * 8
                    pltpu.sync_copy(
                        src.at[pl.ds(start, 8)], idx_v.at[pl.ds(0, 8)]
                    )
                    for r in range(8):
                        scatter_row(r)

            pltpu.sync_copy(hist_v, dst.at[pl.ds(w * NPAD, NPAD)])

    part = jax.ShapeDtypeStruct((NW * NPAD,), jnp.float32)
    kern = functools.partial(
        pl.kernel,
        out_type=(part, part),
        mesh=_mesh(),
        scratch_types=[
            pltpu.VMEM((G, 128), jnp.int32),
            pltpu.VMEM((NPAD,), jnp.float32),
        ],
        compiler_params=pltpu.CompilerParams(needs_layout_passes=False, use_tc_tiling_on_sc=False),
        name="sc_degree_hist",
    )
    return kern(body)


def _make_gs(F, GPT, GTAIL, GG, NCHUNK, NPAD):
    G = GG * 8
    NPT = NPAD // NS

    def body(row2d, col2d, table, zeros_hbm, out, rowb, colb, rows_v, acc_sh,
             gsem, ssem):
        cid = lax.axis_index("c")
        sid = lax.axis_index("s")
        w = sid * NC + cid

        def do_row(j):
            pltpu.async_copy(
                table.at[rowb.at[j]], rows_v.at[pl.ds(j * 128, 128)], gsem
            ).wait()
            pltpu.async_copy(
                rows_v.at[pl.ds(j * 128, 128)], acc_sh.at[colb.at[j]], ssem,
                add=True,
            ).wait()

        def f_body(f, _):
            pltpu.sync_copy(zeros_hbm, acc_sh.at[pl.ds(sid * NPT, NPT)])
            plsc.subcore_barrier()

            def chunk_body(c, _):
                start = (w * GPT + c * GG) * 8
                pltpu.sync_copy(row2d.at[pl.ds(start, G)], rowb)
                pltpu.sync_copy(col2d.at[pl.ds(start, G)], colb)
                if F > 1:
                    off = f * NPAD

                    def add_body(r, _):
                        for k in range(128 // LANE):
                            sl = pl.ds(k * LANE, LANE)
                            rowb[r, sl] = rowb[r, sl] + off
                        return 0

                    lax.fori_loop(0, G, add_body, 0)
                for j in range(G):
                    do_row(j)
                return 0

            lax.fori_loop(0, NCHUNK, chunk_body, 0)

            if GTAIL:
                @pl.when(w < GTAIL)
                def _():
                    start = (GPT * NW + w) * 8
                    pltpu.sync_copy(
                        row2d.at[pl.ds(start, 8)], rowb.at[pl.ds(0, 8)]
                    )
                    pltpu.sync_copy(
                        col2d.at[pl.ds(start, 8)], colb.at[pl.ds(0, 8)]
                    )
                    if F > 1:
                        off = f * NPAD

                        def tadd_body(r, _):
                            for k in range(128 // LANE):
                                sl = pl.ds(k * LANE, LANE)
                                rowb[r, sl] = rowb[r, sl] + off
                            return 0

                        lax.fori_loop(0, 8, tadd_body, 0)
                    for j in range(8):
                        do_row(j)

            plsc.subcore_barrier()
            pltpu.sync_copy(
                acc_sh.at[pl.ds(sid * NPT, NPT)],
                out.at[pl.ds(f * (NC * NPAD) + cid * NPAD + sid * NPT, NPT)],
            )
            plsc.subcore_barrier()
            return 0

        lax.fori_loop(0, F, f_body, 0)

    kern = functools.partial(
        pl.kernel,
        out_type=jax.ShapeDtypeStruct((F * NC * NPAD, 16), jnp.float32),
        mesh=_mesh(),
        scratch_types=[
            pltpu.VMEM((G, 128), jnp.int32),
            pltpu.VMEM((G, 128), jnp.int32),
            pltpu.VMEM((G * 128, 16), jnp.float32),
            pltpu.VMEM_SHARED((NPAD, 16), jnp.float32),
            pltpu.SemaphoreType.DMA,
            pltpu.SemaphoreType.DMA,
        ],
        compiler_params=pltpu.CompilerParams(needs_layout_passes=False, use_tc_tiling_on_sc=False),
        name=f"sc_gather_scatter_f{F}",
    )
    return kern(body)


def _dense_a(src_parts, dst_parts, xr, NPAD, BN):
    grid = (NPAD // BN,)

    def body(sp_ref, dp_ref, x_ref, dinv_ref, indeg_ref, ds_ref, y16t_ref):
        ds = jnp.sum(sp_ref[...], axis=0, keepdims=True)
        indeg = jnp.sum(dp_ref[...], axis=0, keepdims=True)
        dinv = lax.rsqrt(indeg + 1.0)
        xv = x_ref[...]
        y1 = dinv * xv
        y2 = dinv * ds
        dinv_ref[...] = dinv
        indeg_ref[...] = indeg
        ds_ref[...] = ds
        y16t_ref[...] = jnp.concatenate(
            [y1, y2, jnp.zeros((14, BN), jnp.float32)], axis=0
        )

    row = jax.ShapeDtypeStruct((1, NPAD), jnp.float32)
    return pl.pallas_call(
        body,
        grid=grid,
        in_specs=[
            pl.BlockSpec((NW, BN), lambda i: (0, i)),
            pl.BlockSpec((NW, BN), lambda i: (0, i)),
            pl.BlockSpec((1, BN), lambda i: (0, i)),
        ],
        out_specs=[
            pl.BlockSpec((1, BN), lambda i: (0, i)),
            pl.BlockSpec((1, BN), lambda i: (0, i)),
            pl.BlockSpec((1, BN), lambda i: (0, i)),
            pl.BlockSpec((16, BN), lambda i: (0, i)),
        ],
        out_shape=[row, row, row, jax.ShapeDtypeStruct((16, NPAD), jnp.float32)],
    )(src_parts, dst_parts, xr)


def _dense_b(uvt, dinv, xr, dsr, W1T, b1c, NPAD, BN):
    grid = (NPAD // BN,)

    def body(uvt_ref, dinv_ref, x_ref, ds_ref, w1t_ref, b1_ref, a1t_ref):
        uvt_v = uvt_ref[...]
        U = uvt_v[0, 0:1] + uvt_v[1, 0:1]
        V = uvt_v[0, 1:2] + uvt_v[1, 1:2]
        dinv = dinv_ref[...]
        P = dinv * (U + dinv * x_ref[...])
        Q = dinv * (V + dinv * ds_ref[...])
        w = w1t_ref[...]
        pre = w[:, 0:1] * P + w[:, 1:2] * Q + b1_ref[...]
        a1t_ref[...] = jnp.maximum(pre, 0.0)

    return pl.pallas_call(
        body,
        grid=grid,
        in_specs=[
            pl.BlockSpec((NC, 16, BN), lambda i: (0, 0, i)),
            pl.BlockSpec((1, BN), lambda i: (0, i)),
            pl.BlockSpec((1, BN), lambda i: (0, i)),
            pl.BlockSpec((1, BN), lambda i: (0, i)),
            pl.BlockSpec((128, 2), lambda i: (0, 0)),
            pl.BlockSpec((128, 1), lambda i: (0, 0)),
        ],
        out_specs=pl.BlockSpec((128, BN), lambda i: (0, i)),
        out_shape=jax.ShapeDtypeStruct((128, NPAD), jnp.float32),
    )(uvt, dinv, xr, dsr, W1T, b1c)


def _dense_c(nb0t, nb1t, a1t, indeg, W2lT, W2rT, Wv1T, Wv2T, b2lc, bv1c,
             bv2c, N, NPAD, BN):
    grid = (NPAD // BN,)

    def body(nb0_ref, nb1_ref, a1t_ref, indeg_ref, w2l_ref, w2r_ref,
             wv1_ref, wv2_ref, b2l_ref, bv1_ref, bv2_ref, out_ref):
        i = pl.program_id(0)
        cnt = indeg_ref[...]
        recip = 1.0 / jnp.maximum(cnt, 1.0)
        mean_t = (nb0_ref[...] + nb1_ref[...]) * recip
        a1t_v = a1t_ref[...]
        a2t = (
            jnp.dot(w2l_ref[...], mean_t, preferred_element_type=jnp.float32)
            + b2l_ref[...]
            + jnp.dot(w2r_ref[...], a1t_v, preferred_element_type=jnp.float32)
        )
        h = jnp.maximum(
            jnp.dot(wv1_ref[...], a2t, preferred_element_type=jnp.float32)
            + bv1_ref[...],
            0.0,
        )
        v = jnp.dot(wv2_ref[...], h, preferred_element_type=jnp.float32) \
            + bv2_ref[...]
        colid = i * BN + lax.broadcasted_iota(jnp.int32, (1, BN), 1)
        v = jnp.where(colid < N, v, 0.0)
        s = jnp.sum(v).reshape(1, 1)
        prev = jnp.where(i == 0, jnp.zeros((1, 1), jnp.float32), out_ref[...])
        out_ref[...] = prev + s

    return pl.pallas_call(
        body,
        grid=grid,
        in_specs=[
            pl.BlockSpec((128, BN), lambda i: (0, i)),
            pl.BlockSpec((128, BN), lambda i: (0, i)),
            pl.BlockSpec((128, BN), lambda i: (0, i)),
            pl.BlockSpec((1, BN), lambda i: (0, i)),
            pl.BlockSpec((128, 128), lambda i: (0, 0)),
            pl.BlockSpec((128, 128), lambda i: (0, 0)),
            pl.BlockSpec((64, 128), lambda i: (0, 0)),
            pl.BlockSpec((1, 64), lambda i: (0, 0)),
            pl.BlockSpec((128, 1), lambda i: (0, 0)),
            pl.BlockSpec((64, 1), lambda i: (0, 0)),
            pl.BlockSpec((1, 1), lambda i: (0, 0)),
        ],
        out_specs=pl.BlockSpec((1, 1), lambda i: (0, 0)),
        out_shape=jax.ShapeDtypeStruct((1, 1), jnp.float32),
    )(nb0t, nb1t, a1t, indeg, W2lT, W2rT, Wv1T, Wv2T, b2lc, bv1c, bv2c)


def kernel(x, edge_index, W1, b1, W2l, b2l, W2r, Wv1, bv1, Wv2, bv2):
    N = x.shape[0]
    E = edge_index.shape[1]
    assert E % 1024 == 0
    NPAD = ((N + 2047) // 2048) * 2048
    BN = 2048
    ROWS = E // 128
    GRP = ROWS // 8
    GPT = GRP // NW
    GTAIL = GRP % NW
    GG = 1
    NCHUNK = GPT // GG
    F = 8

    row2d = edge_index[0].reshape(ROWS, 128)
    col2d = edge_index[1].reshape(ROWS, 128)
    xr = jnp.pad(x[:, 0], (0, NPAD - N)).reshape(1, NPAD)

    sp, dp = _make_hist(GPT, GTAIL, GG, NCHUNK, NPAD)(row2d, col2d)
    sp = sp.reshape(NW, NPAD)
    dp = dp.reshape(NW, NPAD)
    dinv, indeg, dsr, y16t = _dense_a(sp, dp, xr, NPAD, BN)

    y16 = y16t.T
    zrows = jnp.zeros((NPAD // NS, 16), jnp.float32)
    uv = _make_gs(1, GPT, GTAIL, GG, NCHUNK, NPAD)(row2d, col2d, y16, zrows)
    uvt = uv.reshape(NC, NPAD, 16).transpose(0, 2, 1)

    a1t = _dense_b(uvt, dinv, xr, dsr, W1.T, b1.reshape(128, 1), NPAD, BN)
    a1s = a1t.reshape(F, 16, NPAD).transpose(0, 2, 1).reshape(F * NPAD, 16)

    nbf = _make_gs(F, GPT, GTAIL, GG, NCHUNK, NPAD)(row2d, col2d, a1s, zrows)
    nb4 = nbf.reshape(F, NC, NPAD, 16)
    nb0t = nb4[:, 0].transpose(0, 2, 1).reshape(128, NPAD)
    nb1t = nb4[:, 1].transpose(0, 2, 1).reshape(128, NPAD)

    out = _dense_c(
        nb0t, nb1t, a1t, indeg, W2l.T, W2r.T, Wv1.T, Wv2.T,
        b2l.reshape(128, 1), bv1.reshape(64, 1), bv2.reshape(1, 1),
        N, NPAD, BN,
    )
    return out[0, 0]
